# baseline (speedup 1.0000x reference)
import jax
import jax.numpy as jnp
from jax.experimental import pallas as pl
from jax.experimental.pallas import tpu as pltpu

_B, _S, _D, _E, _K = 4, 2048, 1024, 16, 2
_T = 1024
_SB = _S // _T

_INV_SQRT2 = 0.7071067811865476


def _gelu_exact(v):
    return 0.5 * v * (1.0 + jax.lax.erf(v * _INV_SQRT2))


def _fused_kernel(x_ref, wr1_ref, br1_ref, wr2_ref, br2_ref,
                  wb1_ref, bb1_ref, wb2_ref, bb2_ref, gu_ref,
                  ew_ref, ei_ref, base_ref, pg_ref, scores_ref):
    x = x_ref[0]

    h = jax.lax.dot_general(x, wr1_ref[...], (((1,), (1,)), ((), ())),
                            preferred_element_type=jnp.float32)
    h = _gelu_exact(h + br1_ref[...])

    hb = jax.lax.dot_general(x, wb1_ref[...], (((1,), (1,)), ((), ())),
                             preferred_element_type=jnp.float32)
    hb = _gelu_exact(hb + bb1_ref[...])

    hcat = jnp.concatenate([h, hb], axis=0)
    ho = jax.lax.dot_general(hcat, wr2_ref[...], (((1,), (1,)), ((), ())),
                             preferred_element_type=jnp.float32)
    scores = ho[0:_T, 0:_E] + br2_ref[...]
    scores_ref[0] = scores
    base = ho[_T:2 * _T, _E:_E + 1] + bb2_ref[0, 0]
    base_ref[0] = base

    g = -jnp.log(-jnp.log(gu_ref[0]))
    logits = scores + g
    m = jnp.max(logits, axis=-1, keepdims=True)
    p = jnp.exp(logits - m)
    p = p / jnp.sum(p, axis=-1, keepdims=True)

    idx = jax.lax.broadcasted_iota(jnp.int32, (_T, _E), 1)
    w1 = jnp.max(p, axis=-1, keepdims=True)
    i1 = jnp.min(jnp.where(p == w1, idx, _E), axis=-1, keepdims=True)
    p2 = jnp.where(idx == i1, -1.0, p)
    w2 = jnp.max(p2, axis=-1, keepdims=True)
    i2 = jnp.min(jnp.where(p2 == w2, idx, _E), axis=-1, keepdims=True)

    ew = jnp.concatenate([w1, w2], axis=1)
    ew_ref[0] = ew
    ei_ref[0] = jnp.concatenate([i1, i2], axis=1)
    pg_ref[0] = ew - base


def kernel(x, W_r1, b_r1, W_r2, b_r2, W_b1, b_b1, W_b2, b_b2, gumbel_u):
    whead = jnp.pad(jnp.concatenate([W_r2, W_b2], axis=0), ((0, 7), (0, 0)))

    grid = (_B * _SB,)
    row3 = lambda i: (i // _SB, i % _SB, 0)
    rep2 = lambda i: (0, 0)

    out_shapes = (
        jax.ShapeDtypeStruct((_B, _S, _K), jnp.float32),
        jax.ShapeDtypeStruct((_B, _S, _K), jnp.int32),
        jax.ShapeDtypeStruct((_B, _S, 1), jnp.float32),
        jax.ShapeDtypeStruct((_B, _S, _K), jnp.float32),
        jax.ShapeDtypeStruct((_B, _S, _E), jnp.float32),
    )

    ew, ei, base, pg, scores = pl.pallas_call(
        _fused_kernel,
        grid=grid,
        in_specs=[
            pl.BlockSpec((1, _T, _D), row3),
            pl.BlockSpec((_D, _D), rep2),
            pl.BlockSpec((1, _D), rep2),
            pl.BlockSpec((24, _D), rep2),
            pl.BlockSpec((1, _E), rep2),
            pl.BlockSpec((_D, _D), rep2),
            pl.BlockSpec((1, _D), rep2),
            pl.BlockSpec((1, _D), rep2),
            pl.BlockSpec(memory_space=pltpu.MemorySpace.SMEM),
            pl.BlockSpec((1, _T, _E), row3),
        ],
        out_specs=(
            pl.BlockSpec((1, _T, _K), row3),
            pl.BlockSpec((1, _T, _K), row3),
            pl.BlockSpec((1, _T, 1), row3),
            pl.BlockSpec((1, _T, _K), row3),
            pl.BlockSpec((1, _T, _E), row3),
        ),
        out_shape=out_shapes,
        compiler_params=pltpu.CompilerParams(
            dimension_semantics=("arbitrary",),
        ),
    )(x, W_r1, b_r1.reshape(1, _D), whead, b_r2.reshape(1, _E),
      W_b1, b_b1.reshape(1, _D), W_b2, b_b2.reshape(1, 1), gumbel_u)

    return ew, ei, base.reshape(_B, _S), pg, scores


# speedup vs baseline: 1.2780x; 1.2780x over previous
import jax
import jax.numpy as jnp
from jax.experimental import pallas as pl
from jax.experimental.pallas import tpu as pltpu

_B, _S, _D, _E, _K = 4, 2048, 1024, 16, 2
_T = 1024
_SB = _S // _T

_INV_SQRT2 = 0.7071067811865476


def _gelu_exact(v):
    return 0.5 * v * (1.0 + jax.lax.erf(v * _INV_SQRT2))


def _fused_kernel(x_ref, wr1_ref, br1_ref, wr2_ref, br2_ref,
                  wb1_ref, bb1_ref, wb2_ref, bb2_ref, gu_ref,
                  ew_ref, ei_ref, base_ref, pg_ref, scores_ref):
    x = x_ref[0]

    h = jax.lax.dot_general(x, wr1_ref[...], (((1,), (1,)), ((), ())),
                            preferred_element_type=jnp.float32)
    h = _gelu_exact(h + br1_ref[...])
    scores = jax.lax.dot_general(h, wr2_ref[...], (((1,), (1,)), ((), ())),
                                 preferred_element_type=jnp.float32)
    scores = scores + br2_ref[...]
    scores_ref[0] = scores

    hb = jax.lax.dot_general(x, wb1_ref[...], (((1,), (1,)), ((), ())),
                             preferred_element_type=jnp.float32)
    hb = _gelu_exact(hb + bb1_ref[...])
    basefull = jax.lax.dot_general(hb, wb2_ref[...], (((1,), (1,)), ((), ())),
                                   preferred_element_type=jnp.float32)
    base = basefull[:, 0:1] + bb2_ref[0, 0]
    base_ref[0] = base

    g = -jnp.log(-jnp.log(gu_ref[0]))
    logits = scores + g
    m = jnp.max(logits, axis=-1, keepdims=True)
    p = jnp.exp(logits - m)
    p = p / jnp.sum(p, axis=-1, keepdims=True)

    idx = jax.lax.broadcasted_iota(jnp.int32, (_T, _E), 1)
    w1 = jnp.max(p, axis=-1, keepdims=True)
    i1 = jnp.min(jnp.where(p == w1, idx, _E), axis=-1, keepdims=True)
    p2 = jnp.where(idx == i1, -1.0, p)
    w2 = jnp.max(p2, axis=-1, keepdims=True)
    i2 = jnp.min(jnp.where(p2 == w2, idx, _E), axis=-1, keepdims=True)

    ew = jnp.concatenate([w1, w2], axis=1)
    ew_ref[0] = ew
    ei_ref[0] = jnp.concatenate([i1, i2], axis=1)
    pg_ref[0] = ew - base


def kernel(x, W_r1, b_r1, W_r2, b_r2, W_b1, b_b1, W_b2, b_b2, gumbel_u):
    wb2p = jnp.pad(W_b2, ((0, 127), (0, 0)))

    grid = (_B * _SB,)
    row3 = lambda i: (i // _SB, i % _SB, 0)
    rep2 = lambda i: (0, 0)

    out_shapes = (
        jax.ShapeDtypeStruct((_B, _S, _K), jnp.float32),
        jax.ShapeDtypeStruct((_B, _S, _K), jnp.int32),
        jax.ShapeDtypeStruct((_B, _S, 1), jnp.float32),
        jax.ShapeDtypeStruct((_B, _S, _K), jnp.float32),
        jax.ShapeDtypeStruct((_B, _S, _E), jnp.float32),
    )

    ew, ei, base, pg, scores = pl.pallas_call(
        _fused_kernel,
        grid=grid,
        in_specs=[
            pl.BlockSpec((1, _T, _D), row3),
            pl.BlockSpec((_D, _D), rep2),
            pl.BlockSpec((1, _D), rep2),
            pl.BlockSpec((_E, _D), rep2),
            pl.BlockSpec((1, _E), rep2),
            pl.BlockSpec((_D, _D), rep2),
            pl.BlockSpec((1, _D), rep2),
            pl.BlockSpec((128, _D), rep2),
            pl.BlockSpec(memory_space=pltpu.MemorySpace.SMEM),
            pl.BlockSpec((1, _T, _E), row3),
        ],
        out_specs=(
            pl.BlockSpec((1, _T, _K), row3),
            pl.BlockSpec((1, _T, _K), row3),
            pl.BlockSpec((1, _T, 1), row3),
            pl.BlockSpec((1, _T, _K), row3),
            pl.BlockSpec((1, _T, _E), row3),
        ),
        out_shape=out_shapes,
        compiler_params=pltpu.CompilerParams(
            dimension_semantics=("arbitrary",),
        ),
    )(x, W_r1, b_r1.reshape(1, _D), W_r2, b_r2.reshape(1, _E),
      W_b1, b_b1.reshape(1, _D), wb2p, b_b2.reshape(1, 1), gumbel_u)

    return ew, ei, base.reshape(_B, _S), pg, scores
